# baseline (speedup 1.0000x reference)
import jax
import jax.numpy as jnp
from jax import lax
from jax.experimental import pallas as pl
from jax.experimental.pallas import tpu as pltpu
from jax.experimental.pallas import tpu_sc as plsc

_VOCAB_PAD = 1024
_DIM = 64
_ROW = 128
_NC = 2
_NS = 16
_LANES = 16
_PCHUNK = 128


def _tanh16(x):
    e = jnp.exp(2.0 * x)
    return 1.0 - 2.0 / (e + 1.0)


def _body(ue_hbm, uo_hbm, t_hbm, out_hbm, shared, tload, tbuild, uev, uov,
          buf0, buf1, sem_idx, sem_ge, sem_go, sem_wr):
    c = lax.axis_index("c")
    s = lax.axis_index("s")
    wid = c * _NS + s
    raw_rows = _VOCAB_PAD // _NS
    pairs = uev.shape[0]
    n_chunks = pairs // _PCHUNK

    idx_e = pltpu.async_copy(ue_hbm.at[pl.ds(wid * pairs, pairs)], uev,
                             sem_idx)
    idx_o = pltpu.async_copy(uo_hbm.at[pl.ds(wid * pairs, pairs)], uov,
                             sem_idx)

    pltpu.sync_copy(t_hbm.at[pl.ds(s * raw_rows, raw_rows)], tload)
    zero = jnp.zeros((_LANES,), jnp.float32)

    def build_row(r, _):
        for c4 in range(_DIM // _LANES):
            sl = pl.ds(c4 * _LANES, _LANES)
            sh = pl.ds(_DIM + c4 * _LANES, _LANES)
            x = _tanh16(tload[r, sl])
            tbuild[2 * r, sl] = x
            tbuild[2 * r, sh] = zero
            tbuild[2 * r + 1, sl] = zero
            tbuild[2 * r + 1, sh] = x
        return _

    lax.fori_loop(0, raw_rows, build_row, None)
    pltpu.sync_copy(tbuild,
                    shared.at[pl.ds(s * 2 * raw_rows, 2 * raw_rows)])
    idx_e.wait()
    idx_o.wait()
    plsc.subcore_barrier()

    out_base = wid * pairs
    bufs = (buf0, buf1)

    def gath_e(i):
        return pltpu.async_copy(
            shared.at[uev.at[pl.ds(i * _PCHUNK, _PCHUNK)]],
            bufs[i % 2], sem_ge)

    writes = [None, None]
    ge = gath_e(0)
    for i in range(n_chunks):
        ge.wait()
        go = pltpu.async_copy(
            shared.at[uov.at[pl.ds(i * _PCHUNK, _PCHUNK)]],
            bufs[i % 2], sem_go, add=True)
        if i + 1 < n_chunks:
            if writes[(i + 1) % 2] is not None:
                writes[(i + 1) % 2].wait()
            ge = gath_e(i + 1)
        go.wait()
        writes[i % 2] = pltpu.async_copy(
            bufs[i % 2],
            out_hbm.at[pl.ds(out_base + i * _PCHUNK, _PCHUNK)], sem_wr)
    writes[(n_chunks - 1) % 2].wait()
    writes[n_chunks % 2].wait()


def kernel(input, table):
    b, l = input.shape
    n = b * l
    npair = n // 2
    nw = _NC * _NS
    pairs_per_tile = npair // nw
    v, d = table.shape

    flat2 = input.reshape(npair, 2).astype(jnp.int32)
    ue = (2 * flat2[:, 0]).reshape(npair)
    uo = (2 * flat2[:, 1] + 1).reshape(npair)
    tpad = jnp.zeros((_VOCAB_PAD, _DIM), jnp.float32).at[:v].set(table)

    mesh = plsc.VectorSubcoreMesh(core_axis_name="c", subcore_axis_name="s")
    run = pl.kernel(
        _body,
        out_type=jax.ShapeDtypeStruct((npair, _ROW), jnp.float32),
        mesh=mesh,
        scratch_types=[
            pltpu.VMEM_SHARED((2 * _VOCAB_PAD, _ROW), jnp.float32),
            pltpu.VMEM((_VOCAB_PAD // _NS, _DIM), jnp.float32),
            pltpu.VMEM((2 * _VOCAB_PAD // _NS, _ROW), jnp.float32),
            pltpu.VMEM((pairs_per_tile,), jnp.int32),
            pltpu.VMEM((pairs_per_tile,), jnp.int32),
            pltpu.VMEM((_PCHUNK, _ROW), jnp.float32),
            pltpu.VMEM((_PCHUNK, _ROW), jnp.float32),
            pltpu.SemaphoreType.DMA,
            pltpu.SemaphoreType.DMA,
            pltpu.SemaphoreType.DMA,
            pltpu.SemaphoreType.DMA,
        ],
    )
    out = run(ue, uo, tpad)

    b_blk = 64
    rows_blk = b_blk * (l // 2)

    def _unpack_body(in_ref, out_ref):
        x = in_ref[...]
        rlo = jnp.repeat(x[:, :_DIM], 2, axis=0)
        rhi = jnp.repeat(x[:, _DIM:], 2, axis=0)
        par = lax.broadcasted_iota(jnp.int32, (2 * rows_blk, _DIM), 0) % 2
        z = jnp.where(par == 0, rlo, rhi)
        out_ref[...] = z.reshape(b_blk, l, _DIM)

    unpack = pl.pallas_call(
        _unpack_body,
        grid=(b // b_blk,),
        in_specs=[pl.BlockSpec((rows_blk, _ROW), lambda i: (i, 0))],
        out_specs=pl.BlockSpec((b_blk, l, _DIM), lambda i: (i, 0, 0)),
        out_shape=jax.ShapeDtypeStruct((b, l, _DIM), jnp.float32),
    )
    return unpack(out)


# speedup vs baseline: 2.6892x; 2.6892x over previous
import jax
import jax.numpy as jnp
from jax import lax
from jax.experimental import pallas as pl
from jax.experimental.pallas import tpu as pltpu
from jax.experimental.pallas import tpu_sc as plsc

_VOCAB_PAD = 1024
_DIM = 64
_ROW = 128
_NC = 2
_NS = 16
_LANES = 16
_LPAD = 56


def _tanh16(x):
    e = jnp.exp(2.0 * x)
    return 1.0 - 2.0 / (e + 1.0)


def _body(idx_hbm, t_hbm, out_hbm, shared, tload, tbuild, uv, buf0, buf1,
          sem_idx, sem_ge, sem_wr):
    c = lax.axis_index("c")
    s = lax.axis_index("s")
    wid = c * _NS + s
    raw_rows = _VOCAB_PAD // _NS
    bpt = uv.shape[0]
    bbase = wid * bpt

    idx_cp = pltpu.async_copy(idx_hbm.at[pl.ds(bbase, bpt)], uv, sem_idx)

    pltpu.sync_copy(t_hbm.at[pl.ds(s * raw_rows, raw_rows)], tload)
    zero = jnp.zeros((_LANES,), jnp.float32)

    def build_row(r, _):
        for c4 in range(_DIM // _LANES):
            sl = pl.ds(c4 * _LANES, _LANES)
            sh = pl.ds(_DIM + c4 * _LANES, _LANES)
            tbuild[r, sl] = _tanh16(tload[r, sl])
            tbuild[r, sh] = zero
        return _

    lax.fori_loop(0, raw_rows, build_row, None)
    pltpu.sync_copy(tbuild, shared.at[pl.ds(s * raw_rows, raw_rows)])
    idx_cp.wait()
    plsc.subcore_barrier()

    bufs = (buf0, buf1)

    def dst(i):
        return out_hbm.at[pl.ds((bbase + i) * _LPAD, _LPAD)]

    pltpu.async_copy(shared.at[uv.at[0]], buf0, sem_ge)

    def step(i, p, cur, nxt):
        pltpu.make_async_copy(shared.at[uv.at[i]], cur, sem_ge).wait()
        @pl.when(i > 0)
        def _():
            pltpu.make_async_copy(nxt, dst(i - 1), sem_wr).wait()
        @pl.when(i + 1 < bpt)
        def _():
            pltpu.async_copy(shared.at[uv.at[i + 1]], nxt, sem_ge)
        pltpu.async_copy(cur, dst(i), sem_wr)

    def pair(j, _):
        step(2 * j, 0, buf0, buf1)
        step(2 * j + 1, 1, buf1, buf0)
        return _

    lax.fori_loop(0, bpt // 2, pair, None)
    pltpu.make_async_copy(buf1, dst(bpt - 1), sem_wr).wait()


def kernel(input, table):
    b, l = input.shape
    nw = _NC * _NS
    bpt = b // nw
    v, d = table.shape

    idxp = jnp.pad(input.astype(jnp.int32), ((0, 0), (0, _LPAD - l)))
    tpad = jnp.zeros((_VOCAB_PAD, _DIM), jnp.float32).at[:v].set(table)

    mesh = plsc.VectorSubcoreMesh(core_axis_name="c", subcore_axis_name="s")
    run = pl.kernel(
        _body,
        out_type=jax.ShapeDtypeStruct((b * _LPAD, _ROW), jnp.float32),
        mesh=mesh,
        scratch_types=[
            pltpu.VMEM_SHARED((_VOCAB_PAD, _ROW), jnp.float32),
            pltpu.VMEM((_VOCAB_PAD // _NS, _DIM), jnp.float32),
            pltpu.VMEM((_VOCAB_PAD // _NS, _ROW), jnp.float32),
            pltpu.VMEM((bpt, _LPAD), jnp.int32),
            pltpu.VMEM((_LPAD, _ROW), jnp.float32),
            pltpu.VMEM((_LPAD, _ROW), jnp.float32),
            pltpu.SemaphoreType.DMA,
            pltpu.SemaphoreType.DMA,
            pltpu.SemaphoreType.DMA,
        ],
    )
    out = run(idxp, tpad)
    return out.reshape(b, _LPAD, _ROW)[:, :l, :d]


# speedup vs baseline: 2.8035x; 1.0425x over previous
import jax
import jax.numpy as jnp
from jax import lax
from jax.experimental import pallas as pl
from jax.experimental.pallas import tpu as pltpu
from jax.experimental.pallas import tpu_sc as plsc

_VOCAB_PAD = 1024
_DIM = 64
_ROW = 128
_NC = 2
_NS = 16
_LANES = 16
_LPAD = 56
_NL = 50


def _tanh16(x):
    e = jnp.exp(2.0 * x)
    return 1.0 - 2.0 / (e + 1.0)


def _body(idx_hbm, t_hbm, out_hbm, shared, tload, tbuild, uv, buf0, buf1,
          sem_idx, sem_ge, sem_wr):
    c = lax.axis_index("c")
    s = lax.axis_index("s")
    wid = c * _NS + s
    raw_rows = _VOCAB_PAD // _NS
    bpt = uv.shape[0] // _LPAD
    bbase = wid * bpt

    idx_cp = pltpu.async_copy(
        idx_hbm.at[pl.ds(bbase * _LPAD, bpt * _LPAD)], uv, sem_idx)

    pltpu.sync_copy(t_hbm.at[pl.ds(s * raw_rows, raw_rows)], tload)
    zero = jnp.zeros((_LANES,), jnp.float32)

    def build_row(r, _):
        for c4 in range(_DIM // _LANES):
            sl = pl.ds(c4 * _LANES, _LANES)
            sh = pl.ds(_DIM + c4 * _LANES, _LANES)
            tbuild[r, sl] = _tanh16(tload[r, sl])
            tbuild[r, sh] = zero
        return _

    lax.fori_loop(0, raw_rows, build_row, None)
    pltpu.sync_copy(tbuild, shared.at[pl.ds(s * raw_rows, raw_rows)])
    idx_cp.wait()
    plsc.subcore_barrier()


    def gath(i, buf):
        return pltpu.async_copy(
            shared.at[uv.at[pl.ds(i * _LPAD, _NL)]],
            buf.at[pl.ds(0, _NL)], sem_ge)

    def dst(i):
        return out_hbm.at[pl.ds((bbase + i) * _LPAD, _LPAD)]

    gath(0, buf0)

    def step(i, cur, nxt):
        pltpu.make_async_copy(shared.at[uv.at[pl.ds(i * _LPAD, _NL)]],
                              cur.at[pl.ds(0, _NL)], sem_ge).wait()
        @pl.when(i > 0)
        def _():
            pltpu.make_async_copy(nxt, dst(i - 1), sem_wr).wait()
        @pl.when(i + 1 < bpt)
        def _():
            gath(i + 1, nxt)
        pltpu.async_copy(cur, dst(i), sem_wr)

    def pair(j, _):
        step(2 * j, buf0, buf1)
        step(2 * j + 1, buf1, buf0)
        return _

    lax.fori_loop(0, bpt // 2, pair, None)
    pltpu.make_async_copy(buf1, dst(bpt - 1), sem_wr).wait()


def kernel(input, table):
    b, l = input.shape
    nw = _NC * _NS
    bpt = b // nw
    v, d = table.shape

    idxp = jnp.pad(input.astype(jnp.int32), ((0, 0), (0, _LPAD - l))).reshape(
        b * _LPAD)
    tpad = jnp.zeros((_VOCAB_PAD, _DIM), jnp.float32).at[:v].set(table)

    mesh = plsc.VectorSubcoreMesh(core_axis_name="c", subcore_axis_name="s")
    run = pl.kernel(
        _body,
        out_type=jax.ShapeDtypeStruct((b * _LPAD, _ROW), jnp.float32),
        mesh=mesh,
        scratch_types=[
            pltpu.VMEM_SHARED((_VOCAB_PAD, _ROW), jnp.float32),
            pltpu.VMEM((_VOCAB_PAD // _NS, _DIM), jnp.float32),
            pltpu.VMEM((_VOCAB_PAD // _NS, _ROW), jnp.float32),
            pltpu.VMEM((bpt * _LPAD,), jnp.int32),
            pltpu.VMEM((_LPAD, _ROW), jnp.float32),
            pltpu.VMEM((_LPAD, _ROW), jnp.float32),
            pltpu.SemaphoreType.DMA,
            pltpu.SemaphoreType.DMA,
            pltpu.SemaphoreType.DMA,
        ],
    )
    out = run(idxp, tpad)
    return out.reshape(b, _LPAD, _ROW)[:, :l, :d]


# speedup vs baseline: 2.8796x; 1.0271x over previous
import jax
import jax.numpy as jnp
from jax import lax
from jax.experimental import pallas as pl
from jax.experimental.pallas import tpu as pltpu
from jax.experimental.pallas import tpu_sc as plsc

_VOCAB_PAD = 1024
_DIM = 64
_ROW = 128
_NC = 2
_NS = 16
_LANES = 16
_LPAD = 56
_NL = 50


def _tanh16(x):
    e = jnp.exp(2.0 * x)
    return 1.0 - 2.0 / (e + 1.0)


def _body(idx_hbm, t_hbm, out_hbm, shared, tload, tbuild, uv, buf0, buf1,
          sem_idx, sem_ge, sem_wr):
    c = lax.axis_index("c")
    s = lax.axis_index("s")
    wid = c * _NS + s
    raw_rows = _VOCAB_PAD // _NS
    bpt = uv.shape[0] // _LPAD
    bbase = wid * bpt

    idx_cp = pltpu.async_copy(
        idx_hbm.at[pl.ds(bbase * _LPAD, bpt * _LPAD)], uv, sem_idx)

    pltpu.sync_copy(t_hbm.at[pl.ds(s * raw_rows, raw_rows)], tload)
    zero = jnp.zeros((_LANES,), jnp.float32)

    def build_row(r, _):
        for c4 in range(_DIM // _LANES):
            sl = pl.ds(c4 * _LANES, _LANES)
            sh = pl.ds(_DIM + c4 * _LANES, _LANES)
            tbuild[r, sl] = _tanh16(tload[r, sl])
            tbuild[r, sh] = zero
        return _

    lax.fori_loop(0, raw_rows, build_row, None)
    pltpu.sync_copy(tbuild, shared.at[pl.ds(s * raw_rows, raw_rows)])
    idx_cp.wait()
    plsc.subcore_barrier()

    n_chunks = bpt // 2

    def gath(ch, k, buf):
        return pltpu.async_copy(
            shared.at[uv.at[pl.ds((2 * ch + k) * _LPAD, _NL)]],
            buf.at[pl.ds(k * _LPAD, _NL)], sem_ge)

    def dst(ch):
        return out_hbm.at[pl.ds((bbase + 2 * ch) * _LPAD, 2 * _LPAD)]

    gath(0, 0, buf0)
    gath(0, 1, buf0)

    def step(ch, cur, nxt):
        pltpu.make_async_copy(shared.at[uv.at[pl.ds(2 * ch * _LPAD, _NL)]],
                              cur.at[pl.ds(0, _NL)], sem_ge).wait()
        pltpu.make_async_copy(shared.at[uv.at[pl.ds(2 * ch * _LPAD, _NL)]],
                              cur.at[pl.ds(_LPAD, _NL)], sem_ge).wait()
        @pl.when(ch > 0)
        def _():
            pltpu.make_async_copy(nxt, dst(ch - 1), sem_wr).wait()
        @pl.when(ch + 1 < n_chunks)
        def _():
            gath(ch + 1, 0, nxt)
            gath(ch + 1, 1, nxt)
        pltpu.async_copy(cur, dst(ch), sem_wr)

    def pair(j, _):
        step(2 * j, buf0, buf1)
        step(2 * j + 1, buf1, buf0)
        return _

    lax.fori_loop(0, n_chunks // 2, pair, None)
    pltpu.make_async_copy(buf1, dst(n_chunks - 1), sem_wr).wait()


def kernel(input, table):
    b, l = input.shape
    nw = _NC * _NS
    bpt = b // nw
    v, d = table.shape

    idxp = jnp.pad(input.astype(jnp.int32), ((0, 0), (0, _LPAD - l))).reshape(
        b * _LPAD)
    tpad = jnp.zeros((_VOCAB_PAD, _DIM), jnp.float32).at[:v].set(table)

    mesh = plsc.VectorSubcoreMesh(core_axis_name="c", subcore_axis_name="s")
    run = pl.kernel(
        _body,
        out_type=jax.ShapeDtypeStruct((b * _LPAD, _ROW), jnp.float32),
        mesh=mesh,
        scratch_types=[
            pltpu.VMEM_SHARED((_VOCAB_PAD, _ROW), jnp.float32),
            pltpu.VMEM((_VOCAB_PAD // _NS, _DIM), jnp.float32),
            pltpu.VMEM((_VOCAB_PAD // _NS, _ROW), jnp.float32),
            pltpu.VMEM((bpt * _LPAD,), jnp.int32),
            pltpu.VMEM((2 * _LPAD, _ROW), jnp.float32),
            pltpu.VMEM((2 * _LPAD, _ROW), jnp.float32),
            pltpu.SemaphoreType.DMA,
            pltpu.SemaphoreType.DMA,
            pltpu.SemaphoreType.DMA,
        ],
    )
    out = run(idxp, tpad)
    return out.reshape(b, _LPAD, _ROW)[:, :l, :d]


# speedup vs baseline: 2.9254x; 1.0159x over previous
import jax
import jax.numpy as jnp
from jax import lax
from jax.experimental import pallas as pl
from jax.experimental.pallas import tpu as pltpu
from jax.experimental.pallas import tpu_sc as plsc

_VOCAB_PAD = 1024
_DIM = 64
_ROW = 128
_NC = 2
_NS = 16
_LANES = 16
_LPAD = 56
_NL = 50
_WB = 4


def _tanh16(x):
    e = jnp.exp(2.0 * x)
    return 1.0 - 2.0 / (e + 1.0)


def _body(idx_hbm, t_hbm, out_hbm, shared, tload, tbuild, uv, buf0, buf1,
          sem_idx, sem_ge, sem_wr):
    c = lax.axis_index("c")
    s = lax.axis_index("s")
    wid = c * _NS + s
    raw_rows = _VOCAB_PAD // _NS
    bpt = uv.shape[0] // _LPAD
    bbase = wid * bpt

    idx_cp = pltpu.async_copy(
        idx_hbm.at[pl.ds(bbase * _LPAD, bpt * _LPAD)], uv, sem_idx)

    pltpu.sync_copy(t_hbm.at[pl.ds(s * raw_rows, raw_rows)], tload)
    zero = jnp.zeros((_LANES,), jnp.float32)

    def build_row(r, _):
        for c4 in range(_DIM // _LANES):
            sl = pl.ds(c4 * _LANES, _LANES)
            sh = pl.ds(_DIM + c4 * _LANES, _LANES)
            tbuild[r, sl] = _tanh16(tload[r, sl])
            tbuild[r, sh] = zero
        return _

    lax.fori_loop(0, raw_rows, build_row, None)
    pltpu.sync_copy(tbuild, shared.at[pl.ds(s * raw_rows, raw_rows)])
    idx_cp.wait()
    plsc.subcore_barrier()

    n_chunks = bpt // _WB

    def gath(ch, k, buf):
        return pltpu.async_copy(
            shared.at[uv.at[pl.ds((_WB * ch + k) * _LPAD, _NL)]],
            buf.at[pl.ds(k * _LPAD, _NL)], sem_ge)

    def dst(ch):
        return out_hbm.at[pl.ds((bbase + _WB * ch) * _LPAD, _WB * _LPAD)]

    for k in range(_WB):
        gath(0, k, buf0)

    def step(ch, cur, nxt):
        for k in range(_WB):
            pltpu.make_async_copy(
                shared.at[uv.at[pl.ds(_WB * ch * _LPAD, _NL)]],
                cur.at[pl.ds(k * _LPAD, _NL)], sem_ge).wait()
        @pl.when(ch > 0)
        def _():
            pltpu.make_async_copy(nxt, dst(ch - 1), sem_wr).wait()
        @pl.when(ch + 1 < n_chunks)
        def _():
            for k in range(_WB):
                gath(ch + 1, k, nxt)
        pltpu.async_copy(cur, dst(ch), sem_wr)

    def pair(j, _):
        step(2 * j, buf0, buf1)
        step(2 * j + 1, buf1, buf0)
        return _

    lax.fori_loop(0, n_chunks // 2, pair, None)
    pltpu.make_async_copy(buf1, dst(n_chunks - 1), sem_wr).wait()


def kernel(input, table):
    b, l = input.shape
    nw = _NC * _NS
    bpt = b // nw
    v, d = table.shape

    idxp = jnp.pad(input.astype(jnp.int32), ((0, 0), (0, _LPAD - l))).reshape(
        b * _LPAD)
    tpad = jnp.zeros((_VOCAB_PAD, _DIM), jnp.float32).at[:v].set(table)

    mesh = plsc.VectorSubcoreMesh(core_axis_name="c", subcore_axis_name="s")
    run = pl.kernel(
        _body,
        out_type=jax.ShapeDtypeStruct((b * _LPAD, _ROW), jnp.float32),
        mesh=mesh,
        scratch_types=[
            pltpu.VMEM_SHARED((_VOCAB_PAD, _ROW), jnp.float32),
            pltpu.VMEM((_VOCAB_PAD // _NS, _DIM), jnp.float32),
            pltpu.VMEM((_VOCAB_PAD // _NS, _ROW), jnp.float32),
            pltpu.VMEM((bpt * _LPAD,), jnp.int32),
            pltpu.VMEM((_WB * _LPAD, _ROW), jnp.float32),
            pltpu.VMEM((_WB * _LPAD, _ROW), jnp.float32),
            pltpu.SemaphoreType.DMA,
            pltpu.SemaphoreType.DMA,
            pltpu.SemaphoreType.DMA,
        ],
    )
    out = run(idxp, tpad)
    return out.reshape(b, _LPAD, _ROW)[:, :l, :d]
